# baseline (speedup 1.0000x reference)
import functools

import jax
import jax.numpy as jnp
from jax.experimental import pallas as pl

_K_QQ = 16
_K_QC = 64
_RB = 256
_HI = jax.lax.Precision.HIGHEST


def _monotone_key(x_f32):
    s = jax.lax.bitcast_convert_type(x_f32, jnp.int32)
    return s ^ (jax.lax.shift_right_arithmetic(s, 31) & jnp.int32(0x7FFFFFFF))


def _key_to_float(k_i32):
    s = k_i32 ^ (jax.lax.shift_right_arithmetic(k_i32, 31) & jnp.int32(0x7FFFFFFF))
    return jax.lax.bitcast_convert_type(s, jnp.float32)


def _dist_thresh_kernel(k, q_ref, b_ref, d2_ref, t_ref):
    q = q_ref[...]
    b = b_ref[...]
    q2 = jnp.sum(q * q, axis=1, keepdims=True)
    b2 = jnp.sum(b * b, axis=1)
    qb = jax.lax.dot_general(q, b, (((1,), (1,)), ((), ())),
                             preferred_element_type=jnp.float32, precision=_HI)
    d2 = q2 + b2[None, :] - 2.0 * qb
    d2_ref[...] = d2

    key = _monotone_key(d2)
    ones = jnp.ones((key.shape[1], 8), jnp.float32)
    kf = jnp.float32(k)

    def body(_, lohi):
        lo, hi = lohi
        mid = (lo >> 1) + (hi >> 1) + (lo & hi & 1)
        mask = jnp.where(key <= mid, 1.0, 0.0)
        cnt = jax.lax.dot_general(mask, ones, (((1,), (0,)), ((), ())),
                                  preferred_element_type=jnp.float32)[:, :1]
        ge = cnt >= kf
        return jnp.where(ge, lo, mid + 1), jnp.where(ge, mid, hi)

    lo0 = jnp.full((q.shape[0], 1), -(2**31), jnp.int32)
    hi0 = jnp.full((q.shape[0], 1), 2**31 - 1, jnp.int32)
    _, hi = jax.lax.fori_loop(0, 32, body, (lo0, hi0))
    t_ref[...] = _key_to_float(hi)


def _dist_thresh(qc8, bc8, k):
    n = qc8.shape[0]
    m = bc8.shape[0]
    grid = (n // _RB,)
    return pl.pallas_call(
        functools.partial(_dist_thresh_kernel, k),
        grid=grid,
        in_specs=[
            pl.BlockSpec((_RB, 8), lambda i: (i, 0)),
            pl.BlockSpec((m, 8), lambda i: (0, 0)),
        ],
        out_specs=[
            pl.BlockSpec((_RB, m), lambda i: (i, 0)),
            pl.BlockSpec((_RB, 1), lambda i: (i, 0)),
        ],
        out_shape=[
            jax.ShapeDtypeStruct((n, m), jnp.float32),
            jax.ShapeDtypeStruct((n, 1), jnp.float32),
        ],
    )(qc8, bc8)


def _split_kernel(x_ref, hi_ref, lo_ref):
    x = x_ref[...]
    hi = x.astype(jnp.bfloat16)
    hi_ref[...] = hi
    lo_ref[...] = (x - hi.astype(jnp.float32)).astype(jnp.bfloat16)


def _split(x):
    n, d = x.shape
    return pl.pallas_call(
        _split_kernel,
        grid=(n // _RB,),
        in_specs=[pl.BlockSpec((_RB, d), lambda i: (i, 0))],
        out_specs=[
            pl.BlockSpec((_RB, d), lambda i: (i, 0)),
            pl.BlockSpec((_RB, d), lambda i: (i, 0)),
        ],
        out_shape=[
            jax.ShapeDtypeStruct((n, d), jnp.bfloat16),
            jax.ShapeDtypeStruct((n, d), jnp.bfloat16),
        ],
    )(x)


def _attn_kernel(d2_ref, t_ref, gfhi_ref, gflo_ref, sf_ref, w_ref, out_ref):
    d2 = d2_ref[...]
    mask = jnp.where(d2 <= t_ref[...], 1.0, 0.0)
    maskb = mask.astype(jnp.bfloat16)
    ones = jnp.ones((d2.shape[1], 8), jnp.bfloat16)
    dn = (((1,), (0,)), ((), ()))
    cnt = jax.lax.dot_general(maskb, ones, dn,
                              preferred_element_type=jnp.float32)[:, :1]
    acc = (jax.lax.dot_general(maskb, gfhi_ref[...], dn,
                               preferred_element_type=jnp.float32)
           + jax.lax.dot_general(maskb, gflo_ref[...], dn,
                                 preferred_element_type=jnp.float32))
    mean = acc / cnt
    up = jax.lax.dot_general(mean, w_ref[...], dn,
                             preferred_element_type=jnp.float32, precision=_HI)
    out_ref[...] = sf_ref[...] + up


def _attn_step(d2, t, gfeat_hilo, sfeat, w):
    gfhi, gflo = gfeat_hilo
    n, m = d2.shape
    d = gfhi.shape[1]
    return pl.pallas_call(
        _attn_kernel,
        grid=(n // _RB,),
        in_specs=[
            pl.BlockSpec((_RB, m), lambda i: (i, 0)),
            pl.BlockSpec((_RB, 1), lambda i: (i, 0)),
            pl.BlockSpec((m, d), lambda i: (0, 0)),
            pl.BlockSpec((m, d), lambda i: (0, 0)),
            pl.BlockSpec((_RB, d), lambda i: (i, 0)),
            pl.BlockSpec((d, d), lambda i: (0, 0)),
        ],
        out_specs=pl.BlockSpec((_RB, d), lambda i: (i, 0)),
        out_shape=jax.ShapeDtypeStruct((n, d), jnp.float32),
    )(d2, t, gfhi, gflo, sfeat, w)


def _sim_stats_kernel(tfhi_ref, tflo_ref, afhi_ref, aflo_ref, sim_ref,
                      rmax_ref, rsum_ref, cmax_ref, csum_ref):
    d = tfhi_ref.shape[1]
    dn = (((1,), (1,)), ((), ()))
    tfhi = tfhi_ref[...]
    afhi = afhi_ref[...]
    s = (jax.lax.dot_general(tfhi, afhi, dn,
                             preferred_element_type=jnp.float32)
         + jax.lax.dot_general(tfhi, aflo_ref[...], dn,
                               preferred_element_type=jnp.float32)
         + jax.lax.dot_general(tflo_ref[...], afhi, dn,
                               preferred_element_type=jnp.float32))
    s = s * (1.0 / jnp.sqrt(jnp.float32(d)))
    sim_ref[...] = s
    rmax = jnp.max(s, axis=1, keepdims=True)
    rmax_ref[...] = rmax
    e = jnp.exp(s - rmax)
    ones = jnp.ones((s.shape[1], 8), jnp.float32)
    rsum_ref[...] = jax.lax.dot_general(e, ones, (((1,), (0,)), ((), ())),
                                        preferred_element_type=jnp.float32)[:, :1]
    cmax = jnp.max(s, axis=0)
    cmax_ref[0, 0, :] = cmax
    csum_ref[0, 0, :] = jnp.sum(jnp.exp(s - cmax[None, :]), axis=0)


def _sim_stats(tf_hilo, af_hilo):
    tfhi, tflo = tf_hilo
    afhi, aflo = af_hilo
    n, d = tfhi.shape
    m = afhi.shape[0]
    g = n // _RB
    return pl.pallas_call(
        _sim_stats_kernel,
        grid=(g,),
        in_specs=[
            pl.BlockSpec((_RB, d), lambda i: (i, 0)),
            pl.BlockSpec((_RB, d), lambda i: (i, 0)),
            pl.BlockSpec((m, d), lambda i: (0, 0)),
            pl.BlockSpec((m, d), lambda i: (0, 0)),
        ],
        out_specs=[
            pl.BlockSpec((_RB, m), lambda i: (i, 0)),
            pl.BlockSpec((_RB, 1), lambda i: (i, 0)),
            pl.BlockSpec((_RB, 1), lambda i: (i, 0)),
            pl.BlockSpec((1, 1, m), lambda i: (i, 0, 0)),
            pl.BlockSpec((1, 1, m), lambda i: (i, 0, 0)),
        ],
        out_shape=[
            jax.ShapeDtypeStruct((n, m), jnp.float32),
            jax.ShapeDtypeStruct((n, 1), jnp.float32),
            jax.ShapeDtypeStruct((n, 1), jnp.float32),
            jax.ShapeDtypeStruct((g, 1, m), jnp.float32),
            jax.ShapeDtypeStruct((g, 1, m), jnp.float32),
        ],
    )(tfhi, tflo, afhi, aflo)


def _dual_softmax_kernel(sim_ref, rmax_ref, rsum_ref, cmaxp_ref, csump_ref,
                         out_ref):
    s = sim_ref[...]
    g = cmaxp_ref.shape[0]
    m = cmaxp_ref.shape[2]
    cmaxp = cmaxp_ref[...].reshape(g, m)
    csump = csump_ref[...].reshape(g, m)
    cmax = jnp.max(cmaxp, axis=0)
    csum = jnp.sum(csump * jnp.exp(cmaxp - cmax[None, :]), axis=0)
    num = jnp.exp((s - rmax_ref[...]) + (s - cmax[None, :]))
    out_ref[...] = num / (rsum_ref[...] * csum[None, :])


def _dual_softmax(sim, rmax, rsum, cmaxp, csump):
    n, m = sim.shape
    g = cmaxp.shape[0]
    return pl.pallas_call(
        _dual_softmax_kernel,
        grid=(n // _RB,),
        in_specs=[
            pl.BlockSpec((_RB, m), lambda i: (i, 0)),
            pl.BlockSpec((_RB, 1), lambda i: (i, 0)),
            pl.BlockSpec((_RB, 1), lambda i: (i, 0)),
            pl.BlockSpec((g, 1, m), lambda i: (0, 0, 0)),
            pl.BlockSpec((g, 1, m), lambda i: (0, 0, 0)),
        ],
        out_specs=pl.BlockSpec((_RB, m), lambda i: (i, 0)),
        out_shape=jax.ShapeDtypeStruct((n, m), jnp.float32),
    )(sim, rmax, rsum, cmaxp, csump)


def kernel(anchor_coord, anchor_feat, anchor_offset, target_coord,
           target_feat, target_offset, Wq1, Wc1, Wq2, Wc2):
    tc8 = jnp.pad(target_coord, ((0, 0), (0, 5)))
    ac8 = jnp.pad(anchor_coord, ((0, 0), (0, 5)))

    dtt, t_tt = _dist_thresh(tc8, tc8, _K_QQ)
    dta, t_ta = _dist_thresh(tc8, ac8, _K_QC)
    dat, t_at = _dist_thresh(ac8, tc8, _K_QC)

    tf = target_feat
    af = anchor_feat
    tf_hl = _split(tf)
    af_hl = _split(af)
    for (wq, wc) in ((Wq1, Wc1), (Wq2, Wc2)):
        tf = _attn_step(dtt, t_tt, tf_hl, tf, wq)
        tf = _attn_step(dta, t_ta, af_hl, tf, wc)
        tf_hl = _split(tf)
        af = _attn_step(dat, t_at, tf_hl, af, wc)
        af_hl = _split(af)

    sim, rmax, rsum, cmaxp, csump = _sim_stats(tf_hl, af_hl)
    return _dual_softmax(sim, rmax, rsum, cmaxp, csump)


# speedup vs baseline: 1.1577x; 1.1577x over previous
import functools

import jax
import jax.numpy as jnp
from jax.experimental import pallas as pl

_K_QQ = 16
_K_QC = 64
_RB = 256
_HI = jax.lax.Precision.HIGHEST


def _monotone_key(x_f32):
    s = jax.lax.bitcast_convert_type(x_f32, jnp.int32)
    return s ^ (jax.lax.shift_right_arithmetic(s, 31) & jnp.int32(0x7FFFFFFF))


def _key_to_float(k_i32):
    s = k_i32 ^ (jax.lax.shift_right_arithmetic(k_i32, 31) & jnp.int32(0x7FFFFFFF))
    return jax.lax.bitcast_convert_type(s, jnp.float32)


def _dist_thresh_kernel(k, q_ref, b_ref, d2_ref, t_ref):
    q = q_ref[...]
    b = b_ref[...]
    q2 = jnp.sum(q * q, axis=1, keepdims=True)
    b2 = jnp.sum(b * b, axis=1)
    qb = jax.lax.dot_general(q, b, (((1,), (1,)), ((), ())),
                             preferred_element_type=jnp.float32, precision=_HI)
    d2 = q2 + b2[None, :] - 2.0 * qb
    d2_ref[...] = d2

    key = _monotone_key(d2)
    lo0 = jnp.min(key, axis=1, keepdims=True)
    hi0 = jnp.max(key, axis=1, keepdims=True)

    def body(_, lohi):
        lo, hi = lohi
        mid = (lo >> 1) + (hi >> 1) + (lo & hi & 1)
        cnt = jnp.sum((key <= mid).astype(jnp.int32), axis=1, keepdims=True)
        ge = cnt >= k
        return jnp.where(ge, lo, mid + 1), jnp.where(ge, mid, hi)

    _, hi = jax.lax.fori_loop(0, 32, body, (lo0, hi0))
    t_ref[...] = _key_to_float(hi)


def _dist_thresh(qc8, bc8, k):
    n = qc8.shape[0]
    m = bc8.shape[0]
    grid = (n // _RB,)
    return pl.pallas_call(
        functools.partial(_dist_thresh_kernel, k),
        grid=grid,
        in_specs=[
            pl.BlockSpec((_RB, 8), lambda i: (i, 0)),
            pl.BlockSpec((m, 8), lambda i: (0, 0)),
        ],
        out_specs=[
            pl.BlockSpec((_RB, m), lambda i: (i, 0)),
            pl.BlockSpec((_RB, 1), lambda i: (i, 0)),
        ],
        out_shape=[
            jax.ShapeDtypeStruct((n, m), jnp.float32),
            jax.ShapeDtypeStruct((n, 1), jnp.float32),
        ],
    )(qc8, bc8)


def _split_kernel(x_ref, hi_ref, lo_ref):
    x = x_ref[...]
    hi = x.astype(jnp.bfloat16)
    hi_ref[...] = hi
    lo_ref[...] = (x - hi.astype(jnp.float32)).astype(jnp.bfloat16)


def _split(x):
    n, d = x.shape
    return pl.pallas_call(
        _split_kernel,
        grid=(n // _RB,),
        in_specs=[pl.BlockSpec((_RB, d), lambda i: (i, 0))],
        out_specs=[
            pl.BlockSpec((_RB, d), lambda i: (i, 0)),
            pl.BlockSpec((_RB, d), lambda i: (i, 0)),
        ],
        out_shape=[
            jax.ShapeDtypeStruct((n, d), jnp.bfloat16),
            jax.ShapeDtypeStruct((n, d), jnp.bfloat16),
        ],
    )(x)


def _attn_kernel(d2_ref, t_ref, gfhi_ref, gflo_ref, sf_ref, w_ref, out_ref):
    d2 = d2_ref[...]
    mask = jnp.where(d2 <= t_ref[...], 1.0, 0.0)
    maskb = mask.astype(jnp.bfloat16)
    ones = jnp.ones((d2.shape[1], 8), jnp.bfloat16)
    dn = (((1,), (0,)), ((), ()))
    cnt = jax.lax.dot_general(maskb, ones, dn,
                              preferred_element_type=jnp.float32)[:, :1]
    acc = (jax.lax.dot_general(maskb, gfhi_ref[...], dn,
                               preferred_element_type=jnp.float32)
           + jax.lax.dot_general(maskb, gflo_ref[...], dn,
                                 preferred_element_type=jnp.float32))
    mean = acc / cnt
    up = jax.lax.dot_general(mean, w_ref[...], dn,
                             preferred_element_type=jnp.float32, precision=_HI)
    out_ref[...] = sf_ref[...] + up


def _attn_step(d2, t, gfeat_hilo, sfeat, w):
    gfhi, gflo = gfeat_hilo
    n, m = d2.shape
    d = gfhi.shape[1]
    return pl.pallas_call(
        _attn_kernel,
        grid=(n // _RB,),
        in_specs=[
            pl.BlockSpec((_RB, m), lambda i: (i, 0)),
            pl.BlockSpec((_RB, 1), lambda i: (i, 0)),
            pl.BlockSpec((m, d), lambda i: (0, 0)),
            pl.BlockSpec((m, d), lambda i: (0, 0)),
            pl.BlockSpec((_RB, d), lambda i: (i, 0)),
            pl.BlockSpec((d, d), lambda i: (0, 0)),
        ],
        out_specs=pl.BlockSpec((_RB, d), lambda i: (i, 0)),
        out_shape=jax.ShapeDtypeStruct((n, d), jnp.float32),
    )(d2, t, gfhi, gflo, sfeat, w)


def _sim_stats_kernel(tfhi_ref, tflo_ref, afhi_ref, aflo_ref, sim_ref,
                      rmax_ref, rsum_ref, cmax_ref, csum_ref):
    d = tfhi_ref.shape[1]
    dn = (((1,), (1,)), ((), ()))
    tfhi = tfhi_ref[...]
    afhi = afhi_ref[...]
    s = (jax.lax.dot_general(tfhi, afhi, dn,
                             preferred_element_type=jnp.float32)
         + jax.lax.dot_general(tfhi, aflo_ref[...], dn,
                               preferred_element_type=jnp.float32)
         + jax.lax.dot_general(tflo_ref[...], afhi, dn,
                               preferred_element_type=jnp.float32))
    s = s * (1.0 / jnp.sqrt(jnp.float32(d)))
    sim_ref[...] = s
    rmax = jnp.max(s, axis=1, keepdims=True)
    rmax_ref[...] = rmax
    e = jnp.exp(s - rmax)
    ones = jnp.ones((s.shape[1], 8), jnp.float32)
    rsum_ref[...] = jax.lax.dot_general(e, ones, (((1,), (0,)), ((), ())),
                                        preferred_element_type=jnp.float32)[:, :1]
    cmax = jnp.max(s, axis=0)
    cmax_ref[0, 0, :] = cmax
    csum_ref[0, 0, :] = jnp.sum(jnp.exp(s - cmax[None, :]), axis=0)


def _sim_stats(tf_hilo, af_hilo):
    tfhi, tflo = tf_hilo
    afhi, aflo = af_hilo
    n, d = tfhi.shape
    m = afhi.shape[0]
    g = n // _RB
    return pl.pallas_call(
        _sim_stats_kernel,
        grid=(g,),
        in_specs=[
            pl.BlockSpec((_RB, d), lambda i: (i, 0)),
            pl.BlockSpec((_RB, d), lambda i: (i, 0)),
            pl.BlockSpec((m, d), lambda i: (0, 0)),
            pl.BlockSpec((m, d), lambda i: (0, 0)),
        ],
        out_specs=[
            pl.BlockSpec((_RB, m), lambda i: (i, 0)),
            pl.BlockSpec((_RB, 1), lambda i: (i, 0)),
            pl.BlockSpec((_RB, 1), lambda i: (i, 0)),
            pl.BlockSpec((1, 1, m), lambda i: (i, 0, 0)),
            pl.BlockSpec((1, 1, m), lambda i: (i, 0, 0)),
        ],
        out_shape=[
            jax.ShapeDtypeStruct((n, m), jnp.float32),
            jax.ShapeDtypeStruct((n, 1), jnp.float32),
            jax.ShapeDtypeStruct((n, 1), jnp.float32),
            jax.ShapeDtypeStruct((g, 1, m), jnp.float32),
            jax.ShapeDtypeStruct((g, 1, m), jnp.float32),
        ],
    )(tfhi, tflo, afhi, aflo)


def _dual_softmax_kernel(sim_ref, rmax_ref, rsum_ref, cmaxp_ref, csump_ref,
                         out_ref):
    s = sim_ref[...]
    g = cmaxp_ref.shape[0]
    m = cmaxp_ref.shape[2]
    cmaxp = cmaxp_ref[...].reshape(g, m)
    csump = csump_ref[...].reshape(g, m)
    cmax = jnp.max(cmaxp, axis=0)
    csum = jnp.sum(csump * jnp.exp(cmaxp - cmax[None, :]), axis=0)
    num = jnp.exp((s - rmax_ref[...]) + (s - cmax[None, :]))
    out_ref[...] = num / (rsum_ref[...] * csum[None, :])


def _dual_softmax(sim, rmax, rsum, cmaxp, csump):
    n, m = sim.shape
    g = cmaxp.shape[0]
    return pl.pallas_call(
        _dual_softmax_kernel,
        grid=(n // _RB,),
        in_specs=[
            pl.BlockSpec((_RB, m), lambda i: (i, 0)),
            pl.BlockSpec((_RB, 1), lambda i: (i, 0)),
            pl.BlockSpec((_RB, 1), lambda i: (i, 0)),
            pl.BlockSpec((g, 1, m), lambda i: (0, 0, 0)),
            pl.BlockSpec((g, 1, m), lambda i: (0, 0, 0)),
        ],
        out_specs=pl.BlockSpec((_RB, m), lambda i: (i, 0)),
        out_shape=jax.ShapeDtypeStruct((n, m), jnp.float32),
    )(sim, rmax, rsum, cmaxp, csump)


def kernel(anchor_coord, anchor_feat, anchor_offset, target_coord,
           target_feat, target_offset, Wq1, Wc1, Wq2, Wc2):
    tc8 = jnp.pad(target_coord, ((0, 0), (0, 5)))
    ac8 = jnp.pad(anchor_coord, ((0, 0), (0, 5)))

    dtt, t_tt = _dist_thresh(tc8, tc8, _K_QQ)
    dta, t_ta = _dist_thresh(tc8, ac8, _K_QC)
    dat, t_at = _dist_thresh(ac8, tc8, _K_QC)

    tf = target_feat
    af = anchor_feat
    tf_hl = _split(tf)
    af_hl = _split(af)
    for (wq, wc) in ((Wq1, Wc1), (Wq2, Wc2)):
        tf = _attn_step(dtt, t_tt, tf_hl, tf, wq)
        tf = _attn_step(dta, t_ta, af_hl, tf, wc)
        tf_hl = _split(tf)
        af = _attn_step(dat, t_at, tf_hl, af, wc)
        af_hl = _split(af)

    sim, rmax, rsum, cmaxp, csump = _sim_stats(tf_hl, af_hl)
    return _dual_softmax(sim, rmax, rsum, cmaxp, csump)
